# baseline (speedup 1.0000x reference)
import functools

import jax
import jax.numpy as jnp
from jax import lax
from jax.experimental import pallas as pl
from jax.experimental.pallas import tpu as pltpu
from jax.experimental.pallas import tpu_sc as plsc

N_NODES_C = 10000
N_EDGES_C = 320000
D_C = 128
N_GRAPHS_C = 64

CHUNK = 128
N_CHUNKS = N_EDGES_C // CHUNK
NC, NS = 2, 16
NW = NC * NS
ROW_BLK = 1000


def _segment_sum_sc(hb, edge_index):
    mesh = plsc.VectorSubcoreMesh(core_axis_name="c", subcore_axis_name="s")
    STRIPE = 624

    NPW = N_CHUNKS // NW
    LEFT = N_CHUNKS - NPW * NW
    NB = 2
    UNROLL = 2 * NB
    MAIN = (NPW // UNROLL) * UNROLL

    @functools.partial(
        pl.kernel,
        out_type=jax.ShapeDtypeStruct((NC, N_NODES_C, D_C), jnp.float32),
        mesh=mesh,
        compiler_params=pltpu.CompilerParams(use_tc_tiling_on_sc=False),
        scratch_types=(
            [pltpu.VMEM((2, CHUNK), jnp.int32) for _ in range(2 * NB)] +
            [pltpu.VMEM((CHUNK, D_C // 2), jnp.int32) for _ in range(NB)] +
            [pltpu.VMEM((CHUNK, D_C), jnp.float32) for _ in range(NB)] +
            [pltpu.VMEM_SHARED((N_NODES_C, D_C), jnp.float32)] +
            [pltpu.SemaphoreType.DMA for _ in range(3 * NB)]
        ),
    )
    def seg_kernel(h_hbm, ei_hbm, out_hbm, *rest):
        idx_bufs = rest[0:2 * NB]
        row_bufs = rest[2 * NB:3 * NB]
        f32_bufs = rest[3 * NB:4 * NB]
        acc_sh = rest[4 * NB]
        gsems = rest[4 * NB + 1:5 * NB + 1]
        isems = rest[5 * NB + 1:7 * NB + 1]
        rows0_v = f32_bufs[0]
        c = lax.axis_index("c")
        s = lax.axis_index("s")
        wid = c * NS + s
        lo = wid * NPW

        def zrow(r, carry):
            for l in range(D_C // 16):
                rows0_v[r, pl.ds(l * 16, 16)] = jnp.zeros((16,), jnp.float32)
            return carry
        lax.fori_loop(0, CHUNK, zrow, 0)
        base = s * STRIPE

        @pl.when(s < NS - 1)
        def _():
            def zcp(i, carry):
                pltpu.sync_copy(rows0_v.at[pl.ds(0, 104)],
                                acc_sh.at[pl.ds(base + i * 104, 104)])
                return carry
            lax.fori_loop(0, 6, zcp, 0)

        @pl.when(s == NS - 1)
        def _():
            def zcp(i, carry):
                pltpu.sync_copy(rows0_v.at[pl.ds(0, 128)],
                                acc_sh.at[pl.ds(base + i * 128, 128)])
                return carry
            lax.fori_loop(0, 5, zcp, 0)
        plsc.subcore_barrier()

        def idx_copy(q, j):
            return pltpu.make_async_copy(
                ei_hbm.at[:, pl.ds((lo + j) * CHUNK, CHUNK)], idx_bufs[q],
                isems[q])

        def gather(b, q):
            return pltpu.make_async_copy(h_hbm.at[idx_bufs[q].at[0]],
                                         row_bufs[b], gsems[b])

        def widen(b):
            bfv, fv = row_bufs[b], f32_bufs[b]
            hi_mask = jnp.int32(-65536)

            def crow(r, carry):
                for k in range(D_C // 32):
                    wv = bfv[r, pl.ds(k * 16, 16)]
                    fv[r, pl.ds(k * 32, 16)] = lax.bitcast_convert_type(
                        wv << 16, jnp.float32)
                    fv[r, pl.ds(k * 32 + 16, 16)] = lax.bitcast_convert_type(
                        wv & hi_mask, jnp.float32)
                return carry
            lax.fori_loop(0, CHUNK, crow, 0)

        def scatter(b, q):
            pltpu.sync_copy(f32_bufs[b], acc_sh.at[idx_bufs[q].at[1]],
                            add=True)

        for q in range(2 * NB):
            idx_copy(q, q).start()
        for b in range(NB):
            idx_copy(b, b).wait()
            gather(b, b).start()

        def body(i, carry):
            j0 = i * UNROLL
            for t in range(UNROLL):
                j = j0 + t
                b = t % NB
                gather(b, t).wait()
                widen(b)

                @pl.when(j + NB < NPW)
                def _():
                    qn = (t + NB) % UNROLL
                    idx_copy(qn, j + NB).wait()
                    gather(b, qn).start()
                scatter(b, t)

                @pl.when(j + UNROLL < NPW)
                def _():
                    idx_copy(t, j + UNROLL).start()
            return carry
        lax.fori_loop(0, MAIN // UNROLL, body, 0)

        for j in range(MAIN, NPW):
            t = j % UNROLL
            b = t % NB
            gather(b, t).wait()
            widen(b)
            scatter(b, t)

        @pl.when(wid < LEFT)
        def _():
            off = (NPW * NW + wid) * CHUNK
            pltpu.sync_copy(ei_hbm.at[:, pl.ds(off, CHUNK)], idx_bufs[0])
            pltpu.async_copy(h_hbm.at[idx_bufs[0].at[0]], row_bufs[0],
                             gsems[0]).wait()
            widen(0)
            scatter(0, 0)
        plsc.subcore_barrier()

        @pl.when(s < NS - 1)
        def _():
            pltpu.sync_copy(acc_sh.at[pl.ds(base, STRIPE)],
                            out_hbm.at[c, pl.ds(base, STRIPE)])

        @pl.when(s == NS - 1)
        def _():
            pltpu.sync_copy(acc_sh.at[pl.ds(base, 640)],
                            out_hbm.at[c, pl.ds(base, 640)])

    hb32 = lax.bitcast_convert_type(
        hb.reshape(N_NODES_C, D_C // 2, 2), jnp.int32)
    return seg_kernel(hb32, edge_index)


def _mlp_body(h, W1_ref, b1_ref, W2_ref, b2_ref, last_relu):
    a = jnp.dot(h, W1_ref[...], preferred_element_type=jnp.float32)
    a = jnp.maximum(a + b1_ref[...], 0.0)
    o = jnp.dot(a, W2_ref[...], preferred_element_type=jnp.float32)
    o = o + b2_ref[...]
    if last_relu:
        o = jnp.maximum(o, 0.0)
    return o


def _interleave_cols(o):
    return o.reshape(-1, D_C // 32, 2, 16).swapaxes(2, 3).reshape(-1, D_C)


def _mlp_tc(x, W1, b1, W2, b2, last_relu, parts=None):
    n = x.shape[0]
    grid = (n // ROW_BLK,)
    w_spec = pl.BlockSpec((D_C, D_C), lambda i: (0, 0))
    b_spec = pl.BlockSpec((1, D_C), lambda i: (0, 0))
    in_specs = [pl.BlockSpec((ROW_BLK, D_C), lambda i: (i, 0))]
    args = [x]
    if parts is not None:
        in_specs.append(pl.BlockSpec((NC, ROW_BLK, D_C), lambda i: (0, i, 0)))
        args.append(parts)
    in_specs += [w_spec, b_spec, w_spec, b_spec]
    args += [W1, b1.reshape(1, D_C), W2, b2.reshape(1, D_C)]

    if parts is None:
        def body(x_ref, W1_ref, b1_ref, W2_ref, b2_ref, o_ref, ob_ref):
            o = _mlp_body(x_ref[...], W1_ref, b1_ref, W2_ref, b2_ref,
                          last_relu)
            o_ref[...] = o
            ob_ref[...] = _interleave_cols(o).astype(jnp.bfloat16)
    else:
        def body(x_ref, p_ref, W1_ref, b1_ref, W2_ref, b2_ref, o_ref,
                 ob_ref):
            h = x_ref[...] + p_ref[0] + p_ref[1]
            o = _mlp_body(h, W1_ref, b1_ref, W2_ref, b2_ref, last_relu)
            o_ref[...] = o
            ob_ref[...] = _interleave_cols(o).astype(jnp.bfloat16)

    return pl.pallas_call(
        body,
        grid=grid,
        in_specs=in_specs,
        out_specs=[pl.BlockSpec((ROW_BLK, D_C), lambda i: (i, 0)),
                   pl.BlockSpec((ROW_BLK, D_C), lambda i: (i, 0))],
        out_shape=[jax.ShapeDtypeStruct((n, D_C), jnp.float32),
                   jax.ShapeDtypeStruct((n, D_C), jnp.bfloat16)],
    )(*args)


def _conv_pool_decode_tc(h, parts, cW1, cb1, cW2, cb2, batch3,
                         dec_W1, dec_b1, dec_W2, dec_b2):
    n_blocks = N_NODES_C // ROW_BLK
    w_spec = pl.BlockSpec((D_C, D_C), lambda i: (0, 0))
    b_spec = pl.BlockSpec((1, D_C), lambda i: (0, 0))

    def body(h_ref, p_ref, cW1_ref, cb1_ref, cW2_ref, cb2_ref, b_ref,
             W1_ref, b1_ref, W2_ref, b2_ref, o_ref, acc_ref, cnt_ref):
        i = pl.program_id(0)

        @pl.when(i == 0)
        def _():
            acc_ref[...] = jnp.zeros((N_GRAPHS_C, D_C), jnp.float32)
            cnt_ref[...] = jnp.zeros((N_GRAPHS_C, D_C), jnp.float32)

        hin = h_ref[...] + p_ref[0] + p_ref[1]
        h3 = _mlp_body(hin, cW1_ref, cb1_ref, cW2_ref, cb2_ref, True)

        ids = b_ref[0, 0, :]
        gids = lax.broadcasted_iota(jnp.int32, (N_GRAPHS_C, ROW_BLK), 0)
        onehot = (ids[None, :] == gids).astype(jnp.float32)
        acc_ref[...] += jnp.dot(onehot, h3,
                                preferred_element_type=jnp.float32)
        cnt_ref[...] += jnp.broadcast_to(
            jnp.sum(onehot, axis=1, keepdims=True), (N_GRAPHS_C, D_C))

        @pl.when(i == n_blocks - 1)
        def _():
            pooled = acc_ref[...] / jnp.maximum(cnt_ref[...], 1.0)
            o_ref[...] = _mlp_body(pooled, W1_ref, b1_ref, W2_ref, b2_ref,
                                   False)

    return pl.pallas_call(
        body,
        grid=(n_blocks,),
        in_specs=[
            pl.BlockSpec((ROW_BLK, D_C), lambda i: (i, 0)),
            pl.BlockSpec((NC, ROW_BLK, D_C), lambda i: (0, i, 0)),
            w_spec, b_spec, w_spec, b_spec,
            pl.BlockSpec((1, 1, ROW_BLK), lambda i: (i, 0, 0)),
            w_spec, b_spec, w_spec, b_spec,
        ],
        out_specs=pl.BlockSpec((N_GRAPHS_C, D_C), lambda i: (0, 0)),
        out_shape=jax.ShapeDtypeStruct((N_GRAPHS_C, D_C), jnp.float32),
        scratch_shapes=[
            pltpu.VMEM((N_GRAPHS_C, D_C), jnp.float32),
            pltpu.VMEM((N_GRAPHS_C, D_C), jnp.float32),
        ],
    )(h, parts, cW1, cb1.reshape(1, D_C), cW2, cb2.reshape(1, D_C), batch3,
      dec_W1, dec_b1.reshape(1, D_C), dec_W2, dec_b2.reshape(1, D_C))


def kernel(x, edge_index, batch, enc_W1, enc_b1, enc_W2, enc_b2,
           conv_W1, conv_b1, conv_W2, conv_b2,
           dec_W1, dec_b1, dec_W2, dec_b2):
    n_layers = conv_W1.shape[0]
    h, hb = _mlp_tc(x, enc_W1, enc_b1, enc_W2, enc_b2, last_relu=False)
    for i in range(n_layers - 1):
        parts = _segment_sum_sc(hb, edge_index)
        h, hb = _mlp_tc(h, conv_W1[i], conv_b1[i], conv_W2[i], conv_b2[i],
                        last_relu=True, parts=parts)
    parts = _segment_sum_sc(hb, edge_index)
    batch3 = batch.reshape(N_NODES_C // ROW_BLK, 1, ROW_BLK)
    return _conv_pool_decode_tc(
        h, parts, conv_W1[n_layers - 1], conv_b1[n_layers - 1],
        conv_W2[n_layers - 1], conv_b2[n_layers - 1], batch3,
        dec_W1, dec_b1, dec_W2, dec_b2)


# speedup vs baseline: 1.4211x; 1.4211x over previous
import functools

import jax
import jax.numpy as jnp
from jax import lax
from jax.experimental import pallas as pl
from jax.experimental.pallas import tpu as pltpu
from jax.experimental.pallas import tpu_sc as plsc

N_NODES_C = 10000
N_EDGES_C = 320000
D_C = 128
N_GRAPHS_C = 64

CHUNK = 128
N_CHUNKS = N_EDGES_C // CHUNK
NC, NS = 2, 16
NW = NC * NS
ROW_BLK = 1000


def _segment_sum_sc(hb, edge_index):
    mesh = plsc.VectorSubcoreMesh(core_axis_name="c", subcore_axis_name="s")
    STRIPE = 624

    NPW = N_CHUNKS // NW
    LEFT = N_CHUNKS - NPW * NW
    NB = 2
    UNROLL = 2 * NB
    MAIN = (NPW // UNROLL) * UNROLL

    @functools.partial(
        pl.kernel,
        out_type=jax.ShapeDtypeStruct((NC, N_NODES_C, D_C), jnp.float32),
        mesh=mesh,
        compiler_params=pltpu.CompilerParams(use_tc_tiling_on_sc=False),
        scratch_types=(
            [pltpu.VMEM((2, CHUNK), jnp.int32) for _ in range(2 * NB)] +
            [pltpu.VMEM((CHUNK, D_C // 2), jnp.int32) for _ in range(NB)] +
            [pltpu.VMEM((CHUNK, D_C), jnp.float32) for _ in range(NB)] +
            [pltpu.VMEM_SHARED((N_NODES_C, D_C), jnp.float32)] +
            [pltpu.SemaphoreType.DMA for _ in range(3 * NB)]
        ),
    )
    def seg_kernel(h_hbm, ei_hbm, out_hbm, *rest):
        idx_bufs = rest[0:2 * NB]
        row_bufs = rest[2 * NB:3 * NB]
        f32_bufs = rest[3 * NB:4 * NB]
        acc_sh = rest[4 * NB]
        gsems = rest[4 * NB + 1:5 * NB + 1]
        isems = rest[5 * NB + 1:7 * NB + 1]
        rows0_v = f32_bufs[0]
        c = lax.axis_index("c")
        s = lax.axis_index("s")
        wid = c * NS + s
        lo = wid * NPW

        def zrow(r, carry):
            for l in range(D_C // 16):
                rows0_v[r, pl.ds(l * 16, 16)] = jnp.zeros((16,), jnp.float32)
            return carry
        lax.fori_loop(0, CHUNK, zrow, 0)
        base = s * STRIPE

        @pl.when(s < NS - 1)
        def _():
            def zcp(i, carry):
                pltpu.sync_copy(rows0_v.at[pl.ds(0, 104)],
                                acc_sh.at[pl.ds(base + i * 104, 104)])
                return carry
            lax.fori_loop(0, 6, zcp, 0)

        @pl.when(s == NS - 1)
        def _():
            def zcp(i, carry):
                pltpu.sync_copy(rows0_v.at[pl.ds(0, 128)],
                                acc_sh.at[pl.ds(base + i * 128, 128)])
                return carry
            lax.fori_loop(0, 5, zcp, 0)
        plsc.subcore_barrier()

        def idx_copy(q, j):
            return pltpu.make_async_copy(
                ei_hbm.at[:, pl.ds((lo + j) * CHUNK, CHUNK)], idx_bufs[q],
                isems[q])

        def gather(b, q):
            return pltpu.make_async_copy(h_hbm.at[idx_bufs[q].at[0]],
                                         row_bufs[b], gsems[b])

        def widen(b):
            bfv, fv = row_bufs[b], f32_bufs[b]
            hi_mask = jnp.int32(-65536)

            @plsc.parallel_loop(0, CHUNK, unroll=8)
            def crow(r):
                for k in range(D_C // 32):
                    wv = bfv[r, pl.ds(k * 16, 16)]
                    fv[r, pl.ds(k * 32, 16)] = lax.bitcast_convert_type(
                        wv << 16, jnp.float32)
                    fv[r, pl.ds(k * 32 + 16, 16)] = lax.bitcast_convert_type(
                        wv & hi_mask, jnp.float32)

        def scatter(b, q):
            pltpu.sync_copy(f32_bufs[b], acc_sh.at[idx_bufs[q].at[1]],
                            add=True)

        for q in range(2 * NB):
            idx_copy(q, q).start()
        for b in range(NB):
            idx_copy(b, b).wait()
            gather(b, b).start()

        def body(i, carry):
            j0 = i * UNROLL
            for t in range(UNROLL):
                j = j0 + t
                b = t % NB
                gather(b, t).wait()
                widen(b)

                @pl.when(j + NB < NPW)
                def _():
                    qn = (t + NB) % UNROLL
                    idx_copy(qn, j + NB).wait()
                    gather(b, qn).start()
                scatter(b, t)

                @pl.when(j + UNROLL < NPW)
                def _():
                    idx_copy(t, j + UNROLL).start()
            return carry
        lax.fori_loop(0, MAIN // UNROLL, body, 0)

        for j in range(MAIN, NPW):
            t = j % UNROLL
            b = t % NB
            gather(b, t).wait()
            widen(b)
            scatter(b, t)

        @pl.when(wid < LEFT)
        def _():
            off = (NPW * NW + wid) * CHUNK
            pltpu.sync_copy(ei_hbm.at[:, pl.ds(off, CHUNK)], idx_bufs[0])
            pltpu.async_copy(h_hbm.at[idx_bufs[0].at[0]], row_bufs[0],
                             gsems[0]).wait()
            widen(0)
            scatter(0, 0)
        plsc.subcore_barrier()

        @pl.when(s < NS - 1)
        def _():
            pltpu.sync_copy(acc_sh.at[pl.ds(base, STRIPE)],
                            out_hbm.at[c, pl.ds(base, STRIPE)])

        @pl.when(s == NS - 1)
        def _():
            pltpu.sync_copy(acc_sh.at[pl.ds(base, 640)],
                            out_hbm.at[c, pl.ds(base, 640)])

    hb32 = lax.bitcast_convert_type(
        hb.reshape(N_NODES_C, D_C // 2, 2), jnp.int32)
    return seg_kernel(hb32, edge_index)


def _mlp_body(h, W1_ref, b1_ref, W2_ref, b2_ref, last_relu):
    a = jnp.dot(h, W1_ref[...], preferred_element_type=jnp.float32)
    a = jnp.maximum(a + b1_ref[...], 0.0)
    o = jnp.dot(a, W2_ref[...], preferred_element_type=jnp.float32)
    o = o + b2_ref[...]
    if last_relu:
        o = jnp.maximum(o, 0.0)
    return o


def _interleave_cols(o):
    return o.reshape(-1, D_C // 32, 2, 16).swapaxes(2, 3).reshape(-1, D_C)


def _mlp_tc(x, W1, b1, W2, b2, last_relu, parts=None):
    n = x.shape[0]
    grid = (n // ROW_BLK,)
    w_spec = pl.BlockSpec((D_C, D_C), lambda i: (0, 0))
    b_spec = pl.BlockSpec((1, D_C), lambda i: (0, 0))
    in_specs = [pl.BlockSpec((ROW_BLK, D_C), lambda i: (i, 0))]
    args = [x]
    if parts is not None:
        in_specs.append(pl.BlockSpec((NC, ROW_BLK, D_C), lambda i: (0, i, 0)))
        args.append(parts)
    in_specs += [w_spec, b_spec, w_spec, b_spec]
    args += [W1, b1.reshape(1, D_C), W2, b2.reshape(1, D_C)]

    if parts is None:
        def body(x_ref, W1_ref, b1_ref, W2_ref, b2_ref, o_ref, ob_ref):
            o = _mlp_body(x_ref[...], W1_ref, b1_ref, W2_ref, b2_ref,
                          last_relu)
            o_ref[...] = o
            ob_ref[...] = _interleave_cols(o).astype(jnp.bfloat16)
    else:
        def body(x_ref, p_ref, W1_ref, b1_ref, W2_ref, b2_ref, o_ref,
                 ob_ref):
            h = x_ref[...] + p_ref[0] + p_ref[1]
            o = _mlp_body(h, W1_ref, b1_ref, W2_ref, b2_ref, last_relu)
            o_ref[...] = o
            ob_ref[...] = _interleave_cols(o).astype(jnp.bfloat16)

    return pl.pallas_call(
        body,
        grid=grid,
        in_specs=in_specs,
        out_specs=[pl.BlockSpec((ROW_BLK, D_C), lambda i: (i, 0)),
                   pl.BlockSpec((ROW_BLK, D_C), lambda i: (i, 0))],
        out_shape=[jax.ShapeDtypeStruct((n, D_C), jnp.float32),
                   jax.ShapeDtypeStruct((n, D_C), jnp.bfloat16)],
    )(*args)


def _conv_pool_decode_tc(h, parts, cW1, cb1, cW2, cb2, batch3,
                         dec_W1, dec_b1, dec_W2, dec_b2):
    n_blocks = N_NODES_C // ROW_BLK
    w_spec = pl.BlockSpec((D_C, D_C), lambda i: (0, 0))
    b_spec = pl.BlockSpec((1, D_C), lambda i: (0, 0))

    def body(h_ref, p_ref, cW1_ref, cb1_ref, cW2_ref, cb2_ref, b_ref,
             W1_ref, b1_ref, W2_ref, b2_ref, o_ref, acc_ref, cnt_ref):
        i = pl.program_id(0)

        @pl.when(i == 0)
        def _():
            acc_ref[...] = jnp.zeros((N_GRAPHS_C, D_C), jnp.float32)
            cnt_ref[...] = jnp.zeros((N_GRAPHS_C, D_C), jnp.float32)

        hin = h_ref[...] + p_ref[0] + p_ref[1]
        h3 = _mlp_body(hin, cW1_ref, cb1_ref, cW2_ref, cb2_ref, True)

        ids = b_ref[0, 0, :]
        gids = lax.broadcasted_iota(jnp.int32, (N_GRAPHS_C, ROW_BLK), 0)
        onehot = (ids[None, :] == gids).astype(jnp.float32)
        acc_ref[...] += jnp.dot(onehot, h3,
                                preferred_element_type=jnp.float32)
        cnt_ref[...] += jnp.broadcast_to(
            jnp.sum(onehot, axis=1, keepdims=True), (N_GRAPHS_C, D_C))

        @pl.when(i == n_blocks - 1)
        def _():
            pooled = acc_ref[...] / jnp.maximum(cnt_ref[...], 1.0)
            o_ref[...] = _mlp_body(pooled, W1_ref, b1_ref, W2_ref, b2_ref,
                                   False)

    return pl.pallas_call(
        body,
        grid=(n_blocks,),
        in_specs=[
            pl.BlockSpec((ROW_BLK, D_C), lambda i: (i, 0)),
            pl.BlockSpec((NC, ROW_BLK, D_C), lambda i: (0, i, 0)),
            w_spec, b_spec, w_spec, b_spec,
            pl.BlockSpec((1, 1, ROW_BLK), lambda i: (i, 0, 0)),
            w_spec, b_spec, w_spec, b_spec,
        ],
        out_specs=pl.BlockSpec((N_GRAPHS_C, D_C), lambda i: (0, 0)),
        out_shape=jax.ShapeDtypeStruct((N_GRAPHS_C, D_C), jnp.float32),
        scratch_shapes=[
            pltpu.VMEM((N_GRAPHS_C, D_C), jnp.float32),
            pltpu.VMEM((N_GRAPHS_C, D_C), jnp.float32),
        ],
    )(h, parts, cW1, cb1.reshape(1, D_C), cW2, cb2.reshape(1, D_C), batch3,
      dec_W1, dec_b1.reshape(1, D_C), dec_W2, dec_b2.reshape(1, D_C))


def kernel(x, edge_index, batch, enc_W1, enc_b1, enc_W2, enc_b2,
           conv_W1, conv_b1, conv_W2, conv_b2,
           dec_W1, dec_b1, dec_W2, dec_b2):
    n_layers = conv_W1.shape[0]
    h, hb = _mlp_tc(x, enc_W1, enc_b1, enc_W2, enc_b2, last_relu=False)
    for i in range(n_layers - 1):
        parts = _segment_sum_sc(hb, edge_index)
        h, hb = _mlp_tc(h, conv_W1[i], conv_b1[i], conv_W2[i], conv_b2[i],
                        last_relu=True, parts=parts)
    parts = _segment_sum_sc(hb, edge_index)
    batch3 = batch.reshape(N_NODES_C // ROW_BLK, 1, ROW_BLK)
    return _conv_pool_decode_tc(
        h, parts, conv_W1[n_layers - 1], conv_b1[n_layers - 1],
        conv_W2[n_layers - 1], conv_b2[n_layers - 1], batch3,
        dec_W1, dec_b1, dec_W2, dec_b2)


# speedup vs baseline: 4.0834x; 2.8735x over previous
import functools

import jax
import jax.numpy as jnp
from jax import lax
from jax.experimental import pallas as pl
from jax.experimental.pallas import tpu as pltpu
from jax.experimental.pallas import tpu_sc as plsc

N_NODES_C = 10000
N_EDGES_C = 320000
D_C = 128
N_GRAPHS_C = 64

CHUNK = 128
N_CHUNKS = N_EDGES_C // CHUNK
NC, NS = 2, 16
NW = NC * NS
ROW_BLK = 1000


def _segment_sum_sc(h, edge_index):
    mesh = plsc.VectorSubcoreMesh(core_axis_name="c", subcore_axis_name="s")
    STRIPE = 624

    NPW = N_CHUNKS // NW
    LEFT = N_CHUNKS - NPW * NW
    NB = 3

    @functools.partial(
        pl.kernel,
        out_type=jax.ShapeDtypeStruct((NC, N_NODES_C, D_C), jnp.float32),
        mesh=mesh,
        scratch_types=(
            [pltpu.VMEM((2, CHUNK), jnp.int32) for _ in range(2 * NB)] +
            [pltpu.VMEM((CHUNK, D_C), jnp.float32) for _ in range(NB)] +
            [pltpu.VMEM_SHARED((N_NODES_C, D_C), jnp.float32)] +
            [pltpu.SemaphoreType.DMA for _ in range(3 * NB)]
        ),
    )
    def seg_kernel(h_hbm, ei_hbm, out_hbm, *rest):
        idx_bufs = rest[0:2 * NB]
        row_bufs = rest[2 * NB:3 * NB]
        acc_sh = rest[3 * NB]
        gsems = rest[3 * NB + 1:4 * NB + 1]
        isems = rest[4 * NB + 1:6 * NB + 1]
        rows0_v = row_bufs[0]
        c = lax.axis_index("c")
        s = lax.axis_index("s")
        wid = c * NS + s
        lo = wid * NPW

        def idx_copy(q, j):
            return pltpu.make_async_copy(
                ei_hbm.at[:, pl.ds((lo + j) * CHUNK, CHUNK)], idx_bufs[q],
                isems[q])

        def gather(b, q):
            return pltpu.make_async_copy(h_hbm.at[idx_bufs[q].at[0]],
                                         row_bufs[b], gsems[b])

        def scatter(b, q):
            pltpu.sync_copy(row_bufs[b], acc_sh.at[idx_bufs[q].at[1]],
                            add=True)

        for q in range(2 * NB):
            idx_copy(q, q).start()

        def zrow(r, carry):
            for l in range(D_C // 16):
                rows0_v[r, pl.ds(l * 16, 16)] = jnp.zeros((16,), jnp.float32)
            return carry
        lax.fori_loop(0, CHUNK, zrow, 0)
        base = s * STRIPE

        @pl.when(s < NS - 1)
        def _():
            zd = pltpu.make_async_copy(rows0_v.at[pl.ds(0, 104)],
                                       acc_sh.at[pl.ds(base, 104)], gsems[0])
            for i in range(6):
                pltpu.make_async_copy(
                    rows0_v.at[pl.ds(0, 104)],
                    acc_sh.at[pl.ds(base + i * 104, 104)], gsems[0]).start()
            for i in range(6):
                zd.wait()

        @pl.when(s == NS - 1)
        def _():
            zd = pltpu.make_async_copy(rows0_v.at[pl.ds(0, 128)],
                                       acc_sh.at[pl.ds(base, 128)], gsems[0])
            for i in range(5):
                pltpu.make_async_copy(
                    rows0_v.at[pl.ds(0, 128)],
                    acc_sh.at[pl.ds(base + i * 128, 128)], gsems[0]).start()
            for i in range(5):
                zd.wait()

        for b in range(NB):
            idx_copy(b, b).wait()
            gather(b, b).start()
        plsc.subcore_barrier()

        def body(i, carry):
            j0 = i * (2 * NB)
            for t in range(2 * NB):
                j = j0 + t
                b = t % NB
                gather(b, t).wait()
                scatter(b, t)

                @pl.when(j + 2 * NB < NPW)
                def _():
                    idx_copy(t, j + 2 * NB).start()

                @pl.when(j + NB < NPW)
                def _():
                    qn = (t + NB) % (2 * NB)
                    idx_copy(qn, j + NB).wait()
                    gather(b, qn).start()
            return carry
        lax.fori_loop(0, NPW // (2 * NB), body, 0)

        @pl.when(wid < LEFT)
        def _():
            off = (NPW * NW + wid) * CHUNK
            pltpu.sync_copy(ei_hbm.at[:, pl.ds(off, CHUNK)], idx_bufs[0])
            pltpu.async_copy(h_hbm.at[idx_bufs[0].at[0]], row_bufs[0],
                             gsems[0]).wait()
            scatter(0, 0)
        plsc.subcore_barrier()

        @pl.when(s < NS - 1)
        def _():
            pltpu.sync_copy(acc_sh.at[pl.ds(base, STRIPE)],
                            out_hbm.at[c, pl.ds(base, STRIPE)])

        @pl.when(s == NS - 1)
        def _():
            pltpu.sync_copy(acc_sh.at[pl.ds(base, 640)],
                            out_hbm.at[c, pl.ds(base, 640)])

    return seg_kernel(h, edge_index)


def _mlp_body(h, W1_ref, b1_ref, W2_ref, b2_ref, last_relu):
    a = jnp.dot(h, W1_ref[...], preferred_element_type=jnp.float32)
    a = jnp.maximum(a + b1_ref[...], 0.0)
    o = jnp.dot(a, W2_ref[...], preferred_element_type=jnp.float32)
    o = o + b2_ref[...]
    if last_relu:
        o = jnp.maximum(o, 0.0)
    return o


def _mlp_tc(x, W1, b1, W2, b2, last_relu, parts=None):
    n = x.shape[0]
    grid = (n // ROW_BLK,)
    w_spec = pl.BlockSpec((D_C, D_C), lambda i: (0, 0))
    b_spec = pl.BlockSpec((1, D_C), lambda i: (0, 0))
    in_specs = [pl.BlockSpec((ROW_BLK, D_C), lambda i: (i, 0))]
    args = [x]
    if parts is not None:
        in_specs.append(pl.BlockSpec((NC, ROW_BLK, D_C), lambda i: (0, i, 0)))
        args.append(parts)
    in_specs += [w_spec, b_spec, w_spec, b_spec]
    args += [W1, b1.reshape(1, D_C), W2, b2.reshape(1, D_C)]

    if parts is None:
        def body(x_ref, W1_ref, b1_ref, W2_ref, b2_ref, o_ref):
            o_ref[...] = _mlp_body(x_ref[...], W1_ref, b1_ref, W2_ref, b2_ref,
                                   last_relu)
    else:
        def body(x_ref, p_ref, W1_ref, b1_ref, W2_ref, b2_ref, o_ref):
            h = x_ref[...] + p_ref[0] + p_ref[1]
            o_ref[...] = _mlp_body(h, W1_ref, b1_ref, W2_ref, b2_ref,
                                   last_relu)

    return pl.pallas_call(
        body,
        grid=grid,
        in_specs=in_specs,
        out_specs=pl.BlockSpec((ROW_BLK, D_C), lambda i: (i, 0)),
        out_shape=jax.ShapeDtypeStruct((n, D_C), jnp.float32),
    )(*args)


def _conv_pool_decode_tc(h, parts, cW1, cb1, cW2, cb2, batch3,
                         dec_W1, dec_b1, dec_W2, dec_b2):
    n_blocks = N_NODES_C // ROW_BLK
    w_spec = pl.BlockSpec((D_C, D_C), lambda i: (0, 0))
    b_spec = pl.BlockSpec((1, D_C), lambda i: (0, 0))

    def body(h_ref, p_ref, cW1_ref, cb1_ref, cW2_ref, cb2_ref, b_ref,
             W1_ref, b1_ref, W2_ref, b2_ref, o_ref, acc_ref, cnt_ref):
        i = pl.program_id(0)

        @pl.when(i == 0)
        def _():
            acc_ref[...] = jnp.zeros((N_GRAPHS_C, D_C), jnp.float32)
            cnt_ref[...] = jnp.zeros((N_GRAPHS_C, D_C), jnp.float32)

        hin = h_ref[...] + p_ref[0] + p_ref[1]
        h3 = _mlp_body(hin, cW1_ref, cb1_ref, cW2_ref, cb2_ref, True)

        ids = b_ref[0, 0, :]
        gids = lax.broadcasted_iota(jnp.int32, (N_GRAPHS_C, ROW_BLK), 0)
        onehot = (ids[None, :] == gids).astype(jnp.float32)
        acc_ref[...] += jnp.dot(onehot, h3,
                                preferred_element_type=jnp.float32)
        cnt_ref[...] += jnp.broadcast_to(
            jnp.sum(onehot, axis=1, keepdims=True), (N_GRAPHS_C, D_C))

        @pl.when(i == n_blocks - 1)
        def _():
            pooled = acc_ref[...] / jnp.maximum(cnt_ref[...], 1.0)
            o_ref[...] = _mlp_body(pooled, W1_ref, b1_ref, W2_ref, b2_ref,
                                   False)

    return pl.pallas_call(
        body,
        grid=(n_blocks,),
        in_specs=[
            pl.BlockSpec((ROW_BLK, D_C), lambda i: (i, 0)),
            pl.BlockSpec((NC, ROW_BLK, D_C), lambda i: (0, i, 0)),
            w_spec, b_spec, w_spec, b_spec,
            pl.BlockSpec((1, 1, ROW_BLK), lambda i: (i, 0, 0)),
            w_spec, b_spec, w_spec, b_spec,
        ],
        out_specs=pl.BlockSpec((N_GRAPHS_C, D_C), lambda i: (0, 0)),
        out_shape=jax.ShapeDtypeStruct((N_GRAPHS_C, D_C), jnp.float32),
        scratch_shapes=[
            pltpu.VMEM((N_GRAPHS_C, D_C), jnp.float32),
            pltpu.VMEM((N_GRAPHS_C, D_C), jnp.float32),
        ],
    )(h, parts, cW1, cb1.reshape(1, D_C), cW2, cb2.reshape(1, D_C), batch3,
      dec_W1, dec_b1.reshape(1, D_C), dec_W2, dec_b2.reshape(1, D_C))


def kernel(x, edge_index, batch, enc_W1, enc_b1, enc_W2, enc_b2,
           conv_W1, conv_b1, conv_W2, conv_b2,
           dec_W1, dec_b1, dec_W2, dec_b2):
    n_layers = conv_W1.shape[0]
    h = _mlp_tc(x, enc_W1, enc_b1, enc_W2, enc_b2, last_relu=False)
    for i in range(n_layers - 1):
        parts = _segment_sum_sc(h, edge_index)
        h = _mlp_tc(h, conv_W1[i], conv_b1[i], conv_W2[i], conv_b2[i],
                    last_relu=True, parts=parts)
    parts = _segment_sum_sc(h, edge_index)
    batch3 = batch.reshape(N_NODES_C // ROW_BLK, 1, ROW_BLK)
    return _conv_pool_decode_tc(
        h, parts, conv_W1[n_layers - 1], conv_b1[n_layers - 1],
        conv_W2[n_layers - 1], conv_b2[n_layers - 1], batch3,
        dec_W1, dec_b1, dec_W2, dec_b2)


# speedup vs baseline: 4.2335x; 1.0368x over previous
import functools

import jax
import jax.numpy as jnp
from jax import lax
from jax.experimental import pallas as pl
from jax.experimental.pallas import tpu as pltpu
from jax.experimental.pallas import tpu_sc as plsc

N_NODES_C = 10000
N_EDGES_C = 320000
D_C = 128
N_GRAPHS_C = 64

CHUNK = 128
N_CHUNKS = N_EDGES_C // CHUNK
NC, NS = 2, 16
NW = NC * NS
ROW_BLK = 2000


def _segment_sum_sc(h, edge_index):
    mesh = plsc.VectorSubcoreMesh(core_axis_name="c", subcore_axis_name="s")
    STRIPE = 624

    NPW = N_CHUNKS // NW
    LEFT = N_CHUNKS - NPW * NW
    NB = 3

    @functools.partial(
        pl.kernel,
        out_type=jax.ShapeDtypeStruct((NC, N_NODES_C, D_C), jnp.float32),
        mesh=mesh,
        scratch_types=(
            [pltpu.VMEM((2, CHUNK), jnp.int32) for _ in range(2 * NB)] +
            [pltpu.VMEM((CHUNK, D_C), jnp.float32) for _ in range(NB)] +
            [pltpu.VMEM_SHARED((N_NODES_C, D_C), jnp.float32)] +
            [pltpu.SemaphoreType.DMA for _ in range(3 * NB)]
        ),
    )
    def seg_kernel(h_hbm, ei_hbm, out_hbm, *rest):
        idx_bufs = rest[0:2 * NB]
        row_bufs = rest[2 * NB:3 * NB]
        acc_sh = rest[3 * NB]
        gsems = rest[3 * NB + 1:4 * NB + 1]
        isems = rest[4 * NB + 1:6 * NB + 1]
        rows0_v = row_bufs[0]
        c = lax.axis_index("c")
        s = lax.axis_index("s")
        wid = c * NS + s
        lo = wid * NPW

        def idx_copy(q, j):
            return pltpu.make_async_copy(
                ei_hbm.at[:, pl.ds((lo + j) * CHUNK, CHUNK)], idx_bufs[q],
                isems[q])

        def gather(b, q):
            return pltpu.make_async_copy(h_hbm.at[idx_bufs[q].at[0]],
                                         row_bufs[b], gsems[b])

        def scatter(b, q):
            pltpu.sync_copy(row_bufs[b], acc_sh.at[idx_bufs[q].at[1]],
                            add=True)

        for q in range(2 * NB):
            idx_copy(q, q).start()

        def zrow(r, carry):
            for l in range(D_C // 16):
                rows0_v[r, pl.ds(l * 16, 16)] = jnp.zeros((16,), jnp.float32)
            return carry
        lax.fori_loop(0, CHUNK, zrow, 0)
        base = s * STRIPE

        @pl.when(s < NS - 1)
        def _():
            zd = pltpu.make_async_copy(rows0_v.at[pl.ds(0, 104)],
                                       acc_sh.at[pl.ds(base, 104)], gsems[0])
            for i in range(6):
                pltpu.make_async_copy(
                    rows0_v.at[pl.ds(0, 104)],
                    acc_sh.at[pl.ds(base + i * 104, 104)], gsems[0]).start()
            for i in range(6):
                zd.wait()

        @pl.when(s == NS - 1)
        def _():
            zd = pltpu.make_async_copy(rows0_v.at[pl.ds(0, 128)],
                                       acc_sh.at[pl.ds(base, 128)], gsems[0])
            for i in range(5):
                pltpu.make_async_copy(
                    rows0_v.at[pl.ds(0, 128)],
                    acc_sh.at[pl.ds(base + i * 128, 128)], gsems[0]).start()
            for i in range(5):
                zd.wait()

        for b in range(NB):
            idx_copy(b, b).wait()
            gather(b, b).start()
        plsc.subcore_barrier()

        def body(i, carry):
            j0 = i * (2 * NB)
            for t in range(2 * NB):
                j = j0 + t
                b = t % NB
                gather(b, t).wait()
                scatter(b, t)

                @pl.when(j + 2 * NB < NPW)
                def _():
                    idx_copy(t, j + 2 * NB).start()

                @pl.when(j + NB < NPW)
                def _():
                    qn = (t + NB) % (2 * NB)
                    idx_copy(qn, j + NB).wait()
                    gather(b, qn).start()
            return carry
        lax.fori_loop(0, NPW // (2 * NB), body, 0)

        @pl.when(wid < LEFT)
        def _():
            off = (NPW * NW + wid) * CHUNK
            pltpu.sync_copy(ei_hbm.at[:, pl.ds(off, CHUNK)], idx_bufs[0])
            pltpu.async_copy(h_hbm.at[idx_bufs[0].at[0]], row_bufs[0],
                             gsems[0]).wait()
            scatter(0, 0)
        plsc.subcore_barrier()

        @pl.when(s < NS - 1)
        def _():
            pltpu.sync_copy(acc_sh.at[pl.ds(base, STRIPE)],
                            out_hbm.at[c, pl.ds(base, STRIPE)])

        @pl.when(s == NS - 1)
        def _():
            pltpu.sync_copy(acc_sh.at[pl.ds(base, 640)],
                            out_hbm.at[c, pl.ds(base, 640)])

    return seg_kernel(h, edge_index)


def _mlp_body(h, W1_ref, b1_ref, W2_ref, b2_ref, last_relu):
    a = jnp.dot(h, W1_ref[...], preferred_element_type=jnp.float32)
    a = jnp.maximum(a + b1_ref[...], 0.0)
    o = jnp.dot(a, W2_ref[...], preferred_element_type=jnp.float32)
    o = o + b2_ref[...]
    if last_relu:
        o = jnp.maximum(o, 0.0)
    return o


def _mlp_tc(x, W1, b1, W2, b2, last_relu, parts=None):
    n = x.shape[0]
    grid = (n // ROW_BLK,)
    w_spec = pl.BlockSpec((D_C, D_C), lambda i: (0, 0))
    b_spec = pl.BlockSpec((1, D_C), lambda i: (0, 0))
    in_specs = [pl.BlockSpec((ROW_BLK, D_C), lambda i: (i, 0))]
    args = [x]
    if parts is not None:
        in_specs.append(pl.BlockSpec((NC, ROW_BLK, D_C), lambda i: (0, i, 0)))
        args.append(parts)
    in_specs += [w_spec, b_spec, w_spec, b_spec]
    args += [W1, b1.reshape(1, D_C), W2, b2.reshape(1, D_C)]

    if parts is None:
        def body(x_ref, W1_ref, b1_ref, W2_ref, b2_ref, o_ref):
            o_ref[...] = _mlp_body(x_ref[...], W1_ref, b1_ref, W2_ref, b2_ref,
                                   last_relu)
    else:
        def body(x_ref, p_ref, W1_ref, b1_ref, W2_ref, b2_ref, o_ref):
            h = x_ref[...] + p_ref[0] + p_ref[1]
            o_ref[...] = _mlp_body(h, W1_ref, b1_ref, W2_ref, b2_ref,
                                   last_relu)

    return pl.pallas_call(
        body,
        grid=grid,
        in_specs=in_specs,
        out_specs=pl.BlockSpec((ROW_BLK, D_C), lambda i: (i, 0)),
        out_shape=jax.ShapeDtypeStruct((n, D_C), jnp.float32),
    )(*args)


def _conv_pool_decode_tc(h, parts, cW1, cb1, cW2, cb2, batch3,
                         dec_W1, dec_b1, dec_W2, dec_b2):
    n_blocks = N_NODES_C // ROW_BLK
    w_spec = pl.BlockSpec((D_C, D_C), lambda i: (0, 0))
    b_spec = pl.BlockSpec((1, D_C), lambda i: (0, 0))

    def body(h_ref, p_ref, cW1_ref, cb1_ref, cW2_ref, cb2_ref, b_ref,
             W1_ref, b1_ref, W2_ref, b2_ref, o_ref, acc_ref, cnt_ref):
        i = pl.program_id(0)

        @pl.when(i == 0)
        def _():
            acc_ref[...] = jnp.zeros((N_GRAPHS_C, D_C), jnp.float32)
            cnt_ref[...] = jnp.zeros((N_GRAPHS_C, D_C), jnp.float32)

        hin = h_ref[...] + p_ref[0] + p_ref[1]
        h3 = _mlp_body(hin, cW1_ref, cb1_ref, cW2_ref, cb2_ref, True)

        ids = b_ref[0, 0, :]
        gids = lax.broadcasted_iota(jnp.int32, (N_GRAPHS_C, ROW_BLK), 0)
        onehot = (ids[None, :] == gids).astype(jnp.float32)
        acc_ref[...] += jnp.dot(onehot, h3,
                                preferred_element_type=jnp.float32)
        cnt_ref[...] += jnp.broadcast_to(
            jnp.sum(onehot, axis=1, keepdims=True), (N_GRAPHS_C, D_C))

        @pl.when(i == n_blocks - 1)
        def _():
            pooled = acc_ref[...] / jnp.maximum(cnt_ref[...], 1.0)
            o_ref[...] = _mlp_body(pooled, W1_ref, b1_ref, W2_ref, b2_ref,
                                   False)

    return pl.pallas_call(
        body,
        grid=(n_blocks,),
        in_specs=[
            pl.BlockSpec((ROW_BLK, D_C), lambda i: (i, 0)),
            pl.BlockSpec((NC, ROW_BLK, D_C), lambda i: (0, i, 0)),
            w_spec, b_spec, w_spec, b_spec,
            pl.BlockSpec((1, 1, ROW_BLK), lambda i: (i, 0, 0)),
            w_spec, b_spec, w_spec, b_spec,
        ],
        out_specs=pl.BlockSpec((N_GRAPHS_C, D_C), lambda i: (0, 0)),
        out_shape=jax.ShapeDtypeStruct((N_GRAPHS_C, D_C), jnp.float32),
        scratch_shapes=[
            pltpu.VMEM((N_GRAPHS_C, D_C), jnp.float32),
            pltpu.VMEM((N_GRAPHS_C, D_C), jnp.float32),
        ],
    )(h, parts, cW1, cb1.reshape(1, D_C), cW2, cb2.reshape(1, D_C), batch3,
      dec_W1, dec_b1.reshape(1, D_C), dec_W2, dec_b2.reshape(1, D_C))


def kernel(x, edge_index, batch, enc_W1, enc_b1, enc_W2, enc_b2,
           conv_W1, conv_b1, conv_W2, conv_b2,
           dec_W1, dec_b1, dec_W2, dec_b2):
    n_layers = conv_W1.shape[0]
    h = _mlp_tc(x, enc_W1, enc_b1, enc_W2, enc_b2, last_relu=False)
    for i in range(n_layers - 1):
        parts = _segment_sum_sc(h, edge_index)
        h = _mlp_tc(h, conv_W1[i], conv_b1[i], conv_W2[i], conv_b2[i],
                    last_relu=True, parts=parts)
    parts = _segment_sum_sc(h, edge_index)
    batch3 = batch.reshape(N_NODES_C // ROW_BLK, 1, ROW_BLK)
    return _conv_pool_decode_tc(
        h, parts, conv_W1[n_layers - 1], conv_b1[n_layers - 1],
        conv_W2[n_layers - 1], conv_b2[n_layers - 1], batch3,
        dec_W1, dec_b1, dec_W2, dec_b2)
